# Initial kernel scaffold; baseline (speedup 1.0000x reference)
import jax
import jax.numpy as jnp
from jax.experimental import pallas as pl


def kernel(x, table):
    raise NotImplementedError("write your pallas kernel here")


# speedup vs baseline: 1.1109x; 1.1109x over previous
import functools

import jax
import jax.numpy as jnp
from jax import lax
from jax.experimental import pallas as pl
from jax.experimental.pallas import tpu as pltpu
from jax.experimental.pallas import tpu_sc as plsc

_VOCAB = 1000000
_DIM = 32
_BATCH = 16384
_SEQ = 50

_B = _BATCH * _SEQ
_NW = 32
_B_PER_W = _B // _NW
_CHUNK = 3200
_NCHUNK = _B_PER_W // _CHUNK

_mesh = plsc.VectorSubcoreMesh(core_axis_name="c", subcore_axis_name="s")


@functools.partial(
    pl.kernel,
    out_type=jax.ShapeDtypeStruct((_B, _DIM), jnp.float32),
    mesh=_mesh,
    scratch_types=[
        pltpu.VMEM((_CHUNK,), jnp.int32),
        pltpu.VMEM((_CHUNK, _DIM), jnp.float32),
        pltpu.SemaphoreType.DMA,
    ],
    compiler_params=pltpu.CompilerParams(use_tc_tiling_on_sc=False),
)
def _gather_kernel(idx_hbm, table_hbm, out_hbm, idx_v, rows_v, sem):
    wid = lax.axis_index("s") * 2 + lax.axis_index("c")
    base = wid * _B_PER_W

    def body(i, carry):
        off = pl.multiple_of(base + i * _CHUNK, _CHUNK)
        pltpu.sync_copy(idx_hbm.at[pl.ds(off, _CHUNK)], idx_v)
        pltpu.async_copy(table_hbm.at[idx_v], rows_v, sem).wait()
        pltpu.sync_copy(rows_v, out_hbm.at[pl.ds(off, _CHUNK)])
        return carry

    lax.fori_loop(0, _NCHUNK, body, 0)


def kernel(x, table):
    flat = x.reshape(_B)
    out = _gather_kernel(flat, table)
    return out.reshape(_BATCH, _SEQ, _DIM)[:, None, :, :]


# speedup vs baseline: 1.1120x; 1.0010x over previous
import functools

import jax
import jax.numpy as jnp
from jax import lax
from jax.experimental import pallas as pl
from jax.experimental.pallas import tpu as pltpu
from jax.experimental.pallas import tpu_sc as plsc

_VOCAB = 1000000
_DIM = 32
_BATCH = 16384
_SEQ = 50

_B = _BATCH * _SEQ
_NW = 32
_B_PER_W = _B // _NW
_CHUNK = 1600
_NCHUNK = _B_PER_W // _CHUNK

_mesh = plsc.VectorSubcoreMesh(core_axis_name="c", subcore_axis_name="s")


@functools.partial(
    pl.kernel,
    out_type=jax.ShapeDtypeStruct((_B, _DIM), jnp.float32),
    mesh=_mesh,
    scratch_types=[
        pltpu.VMEM((2, _CHUNK), jnp.int32),
        pltpu.VMEM((2, _CHUNK, _DIM), jnp.float32),
        pltpu.SemaphoreType.DMA((2,)),
        pltpu.SemaphoreType.DMA((2,)),
        pltpu.SemaphoreType.DMA((2,)),
    ],
    compiler_params=pltpu.CompilerParams(use_tc_tiling_on_sc=False),
)
def _gather_kernel(idx_hbm, table_hbm, out_hbm, idx_v, rows_v, sem_i, sem_g,
                   sem_o):
    wid = lax.axis_index("s") * 2 + lax.axis_index("c")
    base = wid * _B_PER_W

    def idx_copy(c, b):
        off = pl.multiple_of(base + c * _CHUNK, _CHUNK)
        return pltpu.make_async_copy(
            idx_hbm.at[pl.ds(off, _CHUNK)], idx_v.at[b], sem_i.at[b])

    def gather(b):
        return pltpu.make_async_copy(
            table_hbm.at[idx_v.at[b]], rows_v.at[b], sem_g.at[b])

    def out_copy(c, b):
        off = pl.multiple_of(base + c * _CHUNK, _CHUNK)
        return pltpu.make_async_copy(
            rows_v.at[b], out_hbm.at[pl.ds(off, _CHUNK)], sem_o.at[b])

    idx_copy(0, 0).start()
    idx_copy(1, 1).start()
    idx_copy(0, 0).wait()
    gather(0).start()

    for c in range(_NCHUNK):
        b = c % 2
        nb = 1 - b
        if c + 1 < _NCHUNK:
            idx_copy(c + 1, nb).wait()
            if c >= 1:
                out_copy(c - 1, nb).wait()
            gather(nb).start()
        gather(b).wait()
        out_copy(c, b).start()
        if c + 2 < _NCHUNK:
            idx_copy(c + 2, b).start()

    out_copy(_NCHUNK - 2, 0 if _NCHUNK % 2 == 0 else 1).wait()
    out_copy(_NCHUNK - 1, 1 if _NCHUNK % 2 == 0 else 0).wait()


def kernel(x, table):
    flat = x.reshape(_B)
    out = _gather_kernel(flat, table)
    return out.reshape(_BATCH, _SEQ, _DIM)[:, None, :, :]


# speedup vs baseline: 1.6417x; 1.4763x over previous
import functools

import jax
import jax.numpy as jnp
from jax import lax
from jax.experimental import pallas as pl
from jax.experimental.pallas import tpu as pltpu
from jax.experimental.pallas import tpu_sc as plsc

_VOCAB = 1000000
_DIM = 32
_BATCH = 16384
_SEQ = 50

_NW = 32
_B = _BATCH * _SEQ

_mesh = plsc.VectorSubcoreMesh(core_axis_name="c", subcore_axis_name="s")


def _wid():
    return lax.axis_index("s") * 2 + lax.axis_index("c")


_K1_CHUNK = 800
_K1_PER_W = _B // _NW
_K1_NCHUNK = _K1_PER_W // _K1_CHUNK


@functools.partial(
    pl.kernel,
    out_type=jax.ShapeDtypeStruct((_B, _DIM), jnp.float32),
    mesh=_mesh,
    scratch_types=[
        pltpu.VMEM((2, _K1_CHUNK), jnp.int32),
        pltpu.VMEM((2, _K1_CHUNK, _DIM), jnp.float32),
        pltpu.SemaphoreType.DMA((2,)),
        pltpu.SemaphoreType.DMA((2,)),
        pltpu.SemaphoreType.DMA((2,)),
    ],
    compiler_params=pltpu.CompilerParams(use_tc_tiling_on_sc=False),
)
def _gather_rows(idx_hbm, table_hbm, out_hbm, idx_v, rows_v, sem_i, sem_g,
                 sem_o):
    base = _wid() * _K1_PER_W

    def idx_copy(c, b):
        off = pl.multiple_of(base + c * _K1_CHUNK, _K1_CHUNK)
        return pltpu.make_async_copy(
            idx_hbm.at[pl.ds(off, _K1_CHUNK)], idx_v.at[b], sem_i.at[b])

    def gather(b):
        return pltpu.make_async_copy(
            table_hbm.at[idx_v.at[b]], rows_v.at[b], sem_g.at[b])

    def out_copy(c, b):
        off = pl.multiple_of(base + c * _K1_CHUNK, _K1_CHUNK)
        return pltpu.make_async_copy(
            rows_v.at[b],
            out_hbm.at[pl.ds(off, _K1_CHUNK)], sem_o.at[b])

    idx_copy(0, 0).start()
    idx_copy(1, 1).start()
    idx_copy(0, 0).wait()
    gather(0).start()

    for c in range(_K1_NCHUNK):
        b = c % 2
        nb = 1 - b
        if c + 1 < _K1_NCHUNK:
            idx_copy(c + 1, nb).wait()
            if c >= 1:
                out_copy(c - 1, nb).wait()
            gather(nb).start()
        gather(b).wait()
        out_copy(c, b).start()
        if c + 2 < _K1_NCHUNK:
            idx_copy(c + 2, b).start()

    out_copy(_K1_NCHUNK - 2, _K1_NCHUNK % 2).wait()
    out_copy(_K1_NCHUNK - 1, 1 - _K1_NCHUNK % 2).wait()


_K2_BB = 8
_K2_ELEMS = _K2_BB * _SEQ * _DIM
_K2_BATCH_PER_W = _BATCH // _NW
_K2_NCHUNK = _K2_BATCH_PER_W // _K2_BB


@functools.partial(
    pl.kernel,
    out_type=jax.ShapeDtypeStruct((_BATCH, _SEQ, _DIM), jnp.float32),
    mesh=_mesh,
    scratch_types=[
        pltpu.VMEM((_K2_ELEMS,), jnp.float32),
        pltpu.VMEM((_K2_BB, _SEQ, _DIM), jnp.float32),
    ],
)
def _format_out(rows_hbm, out_hbm, in_v, out_v):
    bbase = _wid() * _K2_BATCH_PER_W

    def body(ci, carry):
        bs = pl.multiple_of(bbase + ci * _K2_BB, _K2_BB)
        off = pl.multiple_of(bs * _SEQ * _DIM, _K2_ELEMS)
        pltpu.sync_copy(rows_hbm.at[pl.ds(off, _K2_ELEMS)], in_v)

        def bbody(b, carry2):
            fb = b * _SEQ * _DIM
            for l in range(_SEQ):
                for c in range(0, _DIM, 16):
                    o = pl.multiple_of(fb + l * _DIM + c, 16)
                    out_v[b, l, pl.ds(c, 16)] = in_v[pl.ds(o, 16)]
            return carry2

        lax.fori_loop(0, _K2_BB, bbody, 0)

        pltpu.sync_copy(out_v, out_hbm.at[pl.ds(bs, _K2_BB)])
        return carry

    lax.fori_loop(0, _K2_NCHUNK, body, 0)


def kernel(x, table):
    flat = x.reshape(_B)
    rows = _gather_rows(flat, table)
    out = _format_out(rows.reshape(_B * _DIM))
    return out[:, None, :, :]


# speedup vs baseline: 1.7297x; 1.0536x over previous
import functools

import jax
import jax.numpy as jnp
from jax import lax
from jax.experimental import pallas as pl
from jax.experimental.pallas import tpu as pltpu
from jax.experimental.pallas import tpu_sc as plsc

_VOCAB = 1000000
_DIM = 32
_BATCH = 16384
_SEQ = 50

_NW = 32
_B = _BATCH * _SEQ

_mesh = plsc.VectorSubcoreMesh(core_axis_name="c", subcore_axis_name="s")


def _wid():
    return lax.axis_index("s") * 2 + lax.axis_index("c")


_K1_CHUNK = 640
_K1_PER_W = _B // _NW
_K1_NCHUNK = _K1_PER_W // _K1_CHUNK


@functools.partial(
    pl.kernel,
    out_type=jax.ShapeDtypeStruct((_B, _DIM), jnp.float32),
    mesh=_mesh,
    scratch_types=[
        pltpu.VMEM((2, _K1_CHUNK), jnp.int32),
        pltpu.VMEM((2, _K1_CHUNK, _DIM), jnp.float32),
        pltpu.SemaphoreType.DMA((2,)),
        pltpu.SemaphoreType.DMA((2,)),
        pltpu.SemaphoreType.DMA((2,)),
    ],
    compiler_params=pltpu.CompilerParams(use_tc_tiling_on_sc=False),
)
def _gather_rows(idx_hbm, table_hbm, out_hbm, idx_v, rows_v, sem_i, sem_g,
                 sem_o):
    base = _wid() * _K1_PER_W

    def idx_copy(c, b):
        off = pl.multiple_of(base + c * _K1_CHUNK, _K1_CHUNK)
        return pltpu.make_async_copy(
            idx_hbm.at[pl.ds(off, _K1_CHUNK)], idx_v.at[b], sem_i.at[b])

    def gather(b):
        return pltpu.make_async_copy(
            table_hbm.at[idx_v.at[b]], rows_v.at[b], sem_g.at[b])

    def out_copy(c, b):
        off = pl.multiple_of(base + c * _K1_CHUNK, _K1_CHUNK)
        return pltpu.make_async_copy(
            rows_v.at[b],
            out_hbm.at[pl.ds(off, _K1_CHUNK)], sem_o.at[b])

    idx_copy(0, 0).start()
    idx_copy(1, 1).start()
    idx_copy(0, 0).wait()
    gather(0).start()

    for c in range(_K1_NCHUNK):
        b = c % 2
        nb = 1 - b
        if c + 1 < _K1_NCHUNK:
            idx_copy(c + 1, nb).wait()
            if c >= 1:
                out_copy(c - 1, nb).wait()
            gather(nb).start()
        gather(b).wait()
        out_copy(c, b).start()
        if c + 2 < _K1_NCHUNK:
            idx_copy(c + 2, b).start()

    out_copy(_K1_NCHUNK - 2, _K1_NCHUNK % 2).wait()
    out_copy(_K1_NCHUNK - 1, 1 - _K1_NCHUNK % 2).wait()


_K2_BATCH_PER_W = _BATCH // _NW
_K2_BB = 4
_K2_ELEMS = _K2_BB * _SEQ * _DIM
_K2_NBLK = _K2_BATCH_PER_W // _K2_BB


@functools.partial(
    pl.kernel,
    out_type=jax.ShapeDtypeStruct((_BATCH, 1, _SEQ, _DIM), jnp.float32),
    mesh=_mesh,
    scratch_types=[
        pltpu.VMEM((2, _K2_ELEMS), jnp.float32),
        pltpu.VMEM((2, _K2_BB, 1, _SEQ, _DIM), jnp.float32),
        pltpu.SemaphoreType.DMA((2,)),
        pltpu.SemaphoreType.DMA((2,)),
    ],
)
def _format_out(rows_hbm, out_hbm, in_v, blk_v, sem_i, sem_o):
    bbase = _wid() * _K2_BATCH_PER_W

    def in_copy(blk, d):
        off = pl.multiple_of((bbase + blk * _K2_BB) * _SEQ * _DIM,
                             _K2_ELEMS)
        return pltpu.make_async_copy(
            rows_hbm.at[pl.ds(off, _K2_ELEMS)], in_v.at[d], sem_i.at[d])

    def drain(blk, d):
        bs = pl.multiple_of(bbase + blk * _K2_BB, _K2_BB)
        return pltpu.make_async_copy(
            blk_v.at[d], out_hbm.at[pl.ds(bs, _K2_BB)], sem_o.at[d])

    def regcopy(d):
        def bbody(b, carry2):
            fb = b * _SEQ * _DIM
            for g0 in range(0, _SEQ * _DIM // 16, 4):
                vals = []
                for g in range(g0, g0 + 4):
                    o = pl.multiple_of(fb + g * 16, 16)
                    vals.append(in_v[d, pl.ds(o, 16)])
                for i, g in enumerate(range(g0, g0 + 4)):
                    l = (g * 16) // _DIM
                    c = (g * 16) % _DIM
                    blk_v[d, b, 0, l, pl.ds(c, 16)] = vals[i]
            return carry2

        lax.fori_loop(0, _K2_BB, bbody, 0)

    in_copy(0, 0).start()

    def body(blk, carry):
        for d in range(2):
            @pl.when(lax.rem(blk, 2) == d)
            def _():
                nd = 1 - d
                @pl.when(blk + 1 < _K2_NBLK)
                def _():
                    in_copy(blk + 1, nd).start()
                in_copy(blk, d).wait()
                @pl.when(blk >= 2)
                def _():
                    drain(blk - 2, d).wait()
                regcopy(d)
                drain(blk, d).start()
        return carry

    lax.fori_loop(0, _K2_NBLK, body, 0)

    drain(_K2_NBLK - 2, _K2_NBLK % 2).wait()
    drain(_K2_NBLK - 1, 1 - _K2_NBLK % 2).wait()


def kernel(x, table):
    flat = x.reshape(_B)
    rows = _gather_rows(flat, table)
    return _format_out(rows.reshape(_B * _DIM))


# speedup vs baseline: 1.7723x; 1.0246x over previous
import functools

import jax
import jax.numpy as jnp
from jax import lax
from jax.experimental import pallas as pl
from jax.experimental.pallas import tpu as pltpu
from jax.experimental.pallas import tpu_sc as plsc

_VOCAB = 1000000
_DIM = 32
_BATCH = 16384
_SEQ = 50

_NW = 32
_B = _BATCH * _SEQ

_mesh = plsc.VectorSubcoreMesh(core_axis_name="c", subcore_axis_name="s")


def _wid():
    return lax.axis_index("s") * 2 + lax.axis_index("c")


_K0_R = 800
_K0_NBLK = _VOCAB // _K0_R
_K0_KMAX = (_K0_NBLK + _NW - 1) // _NW
_K0_ELEMS = _K0_R * _DIM


@functools.partial(
    pl.kernel,
    out_type=jax.ShapeDtypeStruct((_VOCAB * _DIM,), jnp.float32),
    mesh=_mesh,
    scratch_types=[
        pltpu.VMEM((_K0_R, _DIM), jnp.float32),
        pltpu.VMEM((_K0_ELEMS,), jnp.float32),
        pltpu.SemaphoreType.DMA,
    ],
)
def _flatten_table(table_hbm, tlin_hbm, in_v, out_v, sem):
    w = _wid()

    def body(k, carry):
        c = w + k * _NW

        @pl.when(c < _K0_NBLK)
        def _():
            r0 = pl.multiple_of(c * _K0_R, _K0_R)
            o0 = pl.multiple_of(c * _K0_ELEMS, _K0_ELEMS)
            pltpu.sync_copy(table_hbm.at[pl.ds(r0, _K0_R)], in_v)

            def rbody(r8, carry2):
                rb = r8 * 8
                vals = []
                for rr in range(8):
                    for cc in range(0, _DIM, 16):
                        vals.append((rr, cc))
                for g0 in range(0, 16, 4):
                    vv = [in_v[rb + vals[g][0], pl.ds(vals[g][1], 16)]
                          for g in range(g0, g0 + 4)]
                    for i, g in enumerate(range(g0, g0 + 4)):
                        o = pl.multiple_of(
                            (rb + vals[g][0]) * _DIM + vals[g][1], 16)
                        out_v[pl.ds(o, 16)] = vv[i]
                return carry2

            lax.fori_loop(0, _K0_R // 8, rbody, 0)

            pltpu.make_async_copy(
                out_v, tlin_hbm.at[pl.ds(o0, _K0_ELEMS)], sem).start()
            pltpu.make_async_copy(
                out_v, tlin_hbm.at[pl.ds(o0, _K0_ELEMS)], sem).wait()

        return carry

    lax.fori_loop(0, _K0_KMAX, body, 0)


_K1_CHUNK = 640
_K1_PER_W = _B // _NW
_K1_NCHUNK = _K1_PER_W // _K1_CHUNK


@functools.partial(
    pl.kernel,
    out_type=jax.ShapeDtypeStruct((_B, _DIM), jnp.float32),
    mesh=_mesh,
    scratch_types=[
        pltpu.VMEM((2, _K1_CHUNK), jnp.int32),
        pltpu.VMEM((2, _K1_CHUNK, _DIM), jnp.float32),
        pltpu.SemaphoreType.DMA((2,)),
        pltpu.SemaphoreType.DMA((2,)),
        pltpu.SemaphoreType.DMA((2,)),
    ],
    compiler_params=pltpu.CompilerParams(use_tc_tiling_on_sc=False),
)
def _gather_rows(idx_hbm, table_hbm, out_hbm, idx_v, rows_v, sem_i, sem_g,
                 sem_o):
    base = _wid() * _K1_PER_W

    def idx_copy(c, b):
        off = pl.multiple_of(base + c * _K1_CHUNK, _K1_CHUNK)
        return pltpu.make_async_copy(
            idx_hbm.at[pl.ds(off, _K1_CHUNK)], idx_v.at[b], sem_i.at[b])

    def gather(b):
        return pltpu.make_async_copy(
            table_hbm.at[idx_v.at[b]], rows_v.at[b], sem_g.at[b])

    def out_copy(c, b):
        off = pl.multiple_of(base + c * _K1_CHUNK, _K1_CHUNK)
        return pltpu.make_async_copy(
            rows_v.at[b],
            out_hbm.at[pl.ds(off, _K1_CHUNK)], sem_o.at[b])

    idx_copy(0, 0).start()
    idx_copy(1, 1).start()
    idx_copy(0, 0).wait()
    gather(0).start()

    for c in range(_K1_NCHUNK):
        b = c % 2
        nb = 1 - b
        if c + 1 < _K1_NCHUNK:
            idx_copy(c + 1, nb).wait()
            if c >= 1:
                out_copy(c - 1, nb).wait()
            gather(nb).start()
        gather(b).wait()
        out_copy(c, b).start()
        if c + 2 < _K1_NCHUNK:
            idx_copy(c + 2, b).start()

    out_copy(_K1_NCHUNK - 2, _K1_NCHUNK % 2).wait()
    out_copy(_K1_NCHUNK - 1, 1 - _K1_NCHUNK % 2).wait()


_K2_BATCH_PER_W = _BATCH // _NW
_K2_BB = 4
_K2_ELEMS = _K2_BB * _SEQ * _DIM
_K2_NBLK = _K2_BATCH_PER_W // _K2_BB


@functools.partial(
    pl.kernel,
    out_type=jax.ShapeDtypeStruct((_BATCH, _SEQ, _DIM), jnp.float32),
    mesh=_mesh,
    scratch_types=[
        pltpu.VMEM((2, _K2_ELEMS), jnp.float32),
        pltpu.VMEM((2, _K2_BB, _SEQ, _DIM), jnp.float32),
        pltpu.SemaphoreType.DMA((2,)),
        pltpu.SemaphoreType.DMA((2,)),
    ],
)
def _format_out(rows_hbm, out_hbm, in_v, blk_v, sem_i, sem_o):
    bbase = _wid() * _K2_BATCH_PER_W

    def in_copy(blk, d):
        off = pl.multiple_of((bbase + blk * _K2_BB) * _SEQ * _DIM,
                             _K2_ELEMS)
        return pltpu.make_async_copy(
            rows_hbm.at[pl.ds(off, _K2_ELEMS)], in_v.at[d], sem_i.at[d])

    def drain(blk, d):
        bs = pl.multiple_of(bbase + blk * _K2_BB, _K2_BB)
        return pltpu.make_async_copy(
            blk_v.at[d], out_hbm.at[pl.ds(bs, _K2_BB)], sem_o.at[d])

    def regcopy(d):
        def bbody(b, carry2):
            fb = b * _SEQ * _DIM
            for g0 in range(0, _SEQ * _DIM // 16, 4):
                vals = []
                for g in range(g0, g0 + 4):
                    o = pl.multiple_of(fb + g * 16, 16)
                    vals.append(in_v[d, pl.ds(o, 16)])
                for i, g in enumerate(range(g0, g0 + 4)):
                    l = (g * 16) // _DIM
                    c = (g * 16) % _DIM
                    blk_v[d, b, l, pl.ds(c, 16)] = vals[i]
            return carry2

        lax.fori_loop(0, _K2_BB, bbody, 0)

    in_copy(0, 0).start()

    def body(blk, carry):
        for d in range(2):
            @pl.when(lax.rem(blk, 2) == d)
            def _():
                nd = 1 - d
                @pl.when(blk + 1 < _K2_NBLK)
                def _():
                    in_copy(blk + 1, nd).start()
                in_copy(blk, d).wait()
                @pl.when(blk >= 2)
                def _():
                    drain(blk - 2, d).wait()
                regcopy(d)
                drain(blk, d).start()
        return carry

    lax.fori_loop(0, _K2_NBLK, body, 0)

    drain(_K2_NBLK - 2, _K2_NBLK % 2).wait()
    drain(_K2_NBLK - 1, 1 - _K2_NBLK % 2).wait()


def kernel(x, table):
    flat = x.reshape(_B)
    tlin = _flatten_table(table)
    rows = _gather_rows(flat, tlin.reshape(_VOCAB, _DIM))
    out = _format_out(rows.reshape(_B * _DIM))
    return out[:, None, :, :]


# speedup vs baseline: 1.9479x; 1.0991x over previous
import functools

import jax
import jax.numpy as jnp
from jax import lax
from jax.experimental import pallas as pl
from jax.experimental.pallas import tpu as pltpu
from jax.experimental.pallas import tpu_sc as plsc

_VOCAB = 1000000
_DIM = 32
_BATCH = 16384
_SEQ = 50

_NW = 32
_B = _BATCH * _SEQ

_mesh = plsc.VectorSubcoreMesh(core_axis_name="c", subcore_axis_name="s")


def _wid():
    return lax.axis_index("s") * 2 + lax.axis_index("c")


_K0_R = 400
_K0_NBLK = _VOCAB // _K0_R
_K0_KMAX = (_K0_NBLK + _NW - 1) // _NW
_K0_ELEMS = _K0_R * _DIM


@functools.partial(
    pl.kernel,
    out_type=jax.ShapeDtypeStruct((_VOCAB * _DIM,), jnp.float32),
    mesh=_mesh,
    scratch_types=[
        pltpu.VMEM((2, _K0_R, _DIM), jnp.float32),
        pltpu.VMEM((2, _K0_ELEMS), jnp.float32),
        pltpu.SemaphoreType.DMA((2,)),
        pltpu.SemaphoreType.DMA((2,)),
    ],
)
def _flatten_table(table_hbm, tlin_hbm, in_v, out_v, sem_i, sem_o):
    w = _wid()

    def in_copy(k, d):
        c = w + k * _NW
        r0 = pl.multiple_of(c * _K0_R, _K0_R)
        return pltpu.make_async_copy(
            table_hbm.at[pl.ds(r0, _K0_R)], in_v.at[d], sem_i.at[d])

    def out_copy(k, d):
        c = w + k * _NW
        o0 = pl.multiple_of(c * _K0_ELEMS, _K0_ELEMS)
        return pltpu.make_async_copy(
            out_v.at[d], tlin_hbm.at[pl.ds(o0, _K0_ELEMS)], sem_o.at[d])

    def regcopy(d):
        def rbody(r16, carry2):
            rb = r16 * 16
            for sub in range(8):
                rr0 = sub * 2
                vv = []
                for q in range(4):
                    rr = rr0 + q // 2
                    cc = (q % 2) * 16
                    vv.append(in_v[d, rb + rr, pl.ds(cc, 16)])
                for q in range(4):
                    rr = rr0 + q // 2
                    cc = (q % 2) * 16
                    o = pl.multiple_of((rb + rr) * _DIM + cc, 16)
                    out_v[d, pl.ds(o, 16)] = vv[q]
            return carry2

        lax.fori_loop(0, _K0_R // 16, rbody, 0)

    @pl.when(w < _K0_NBLK)
    def _():
        in_copy(0, 0).start()

    def body(k, carry):
        c = w + k * _NW

        @pl.when(c < _K0_NBLK)
        def _():
            for d in range(2):
                @pl.when(lax.rem(k, 2) == d)
                def _():
                    nd = 1 - d
                    @pl.when(w + (k + 1) * _NW < _K0_NBLK)
                    def _():
                        in_copy(k + 1, nd).start()
                    in_copy(k, d).wait()
                    @pl.when(k >= 2)
                    def _():
                        out_copy(k - 2, d).wait()
                    regcopy(d)
                    out_copy(k, d).start()
        return carry

    lax.fori_loop(0, _K0_KMAX, body, 0)

    def tail(k):
        @pl.when((w + k * _NW < _K0_NBLK)
                 & (w + (k + 2) * _NW >= _K0_NBLK))
        def _():
            out_copy(k, k % 2).wait()

    tail(_K0_KMAX - 3)
    tail(_K0_KMAX - 2)
    tail(_K0_KMAX - 1)


_K1_CHUNK = 640
_K1_PER_W = _B // _NW
_K1_NCHUNK = _K1_PER_W // _K1_CHUNK


@functools.partial(
    pl.kernel,
    out_type=jax.ShapeDtypeStruct((_B, _DIM), jnp.float32),
    mesh=_mesh,
    scratch_types=[
        pltpu.VMEM((2, _K1_CHUNK), jnp.int32),
        pltpu.VMEM((2, _K1_CHUNK, _DIM), jnp.float32),
        pltpu.SemaphoreType.DMA((2,)),
        pltpu.SemaphoreType.DMA((2,)),
        pltpu.SemaphoreType.DMA((2,)),
    ],
    compiler_params=pltpu.CompilerParams(use_tc_tiling_on_sc=False),
)
def _gather_rows(idx_hbm, table_hbm, out_hbm, idx_v, rows_v, sem_i, sem_g,
                 sem_o):
    base = _wid() * _K1_PER_W

    def idx_copy(c, b):
        off = pl.multiple_of(base + c * _K1_CHUNK, _K1_CHUNK)
        return pltpu.make_async_copy(
            idx_hbm.at[pl.ds(off, _K1_CHUNK)], idx_v.at[b], sem_i.at[b])

    def gather(b):
        return pltpu.make_async_copy(
            table_hbm.at[idx_v.at[b]], rows_v.at[b], sem_g.at[b])

    def out_copy(c, b):
        off = pl.multiple_of(base + c * _K1_CHUNK, _K1_CHUNK)
        return pltpu.make_async_copy(
            rows_v.at[b],
            out_hbm.at[pl.ds(off, _K1_CHUNK)], sem_o.at[b])

    idx_copy(0, 0).start()
    idx_copy(1, 1).start()
    idx_copy(0, 0).wait()
    gather(0).start()

    for c in range(_K1_NCHUNK):
        b = c % 2
        nb = 1 - b
        if c + 1 < _K1_NCHUNK:
            idx_copy(c + 1, nb).wait()
            if c >= 1:
                out_copy(c - 1, nb).wait()
            gather(nb).start()
        gather(b).wait()
        out_copy(c, b).start()
        if c + 2 < _K1_NCHUNK:
            idx_copy(c + 2, b).start()

    out_copy(_K1_NCHUNK - 2, _K1_NCHUNK % 2).wait()
    out_copy(_K1_NCHUNK - 1, 1 - _K1_NCHUNK % 2).wait()


_K2_BATCH_PER_W = _BATCH // _NW
_K2_BB = 4
_K2_ELEMS = _K2_BB * _SEQ * _DIM
_K2_NBLK = _K2_BATCH_PER_W // _K2_BB


@functools.partial(
    pl.kernel,
    out_type=jax.ShapeDtypeStruct((_BATCH, _SEQ, _DIM), jnp.float32),
    mesh=_mesh,
    scratch_types=[
        pltpu.VMEM((2, _K2_ELEMS), jnp.float32),
        pltpu.VMEM((2, _K2_BB, _SEQ, _DIM), jnp.float32),
        pltpu.SemaphoreType.DMA((2,)),
        pltpu.SemaphoreType.DMA((2,)),
    ],
)
def _format_out(rows_hbm, out_hbm, in_v, blk_v, sem_i, sem_o):
    bbase = _wid() * _K2_BATCH_PER_W

    def in_copy(blk, d):
        off = pl.multiple_of((bbase + blk * _K2_BB) * _SEQ * _DIM,
                             _K2_ELEMS)
        return pltpu.make_async_copy(
            rows_hbm.at[pl.ds(off, _K2_ELEMS)], in_v.at[d], sem_i.at[d])

    def drain(blk, d):
        bs = pl.multiple_of(bbase + blk * _K2_BB, _K2_BB)
        return pltpu.make_async_copy(
            blk_v.at[d], out_hbm.at[pl.ds(bs, _K2_BB)], sem_o.at[d])

    def regcopy(d):
        def bbody(b, carry2):
            fb = b * _SEQ * _DIM
            for g0 in range(0, _SEQ * _DIM // 16, 4):
                vals = []
                for g in range(g0, g0 + 4):
                    o = pl.multiple_of(fb + g * 16, 16)
                    vals.append(in_v[d, pl.ds(o, 16)])
                for i, g in enumerate(range(g0, g0 + 4)):
                    l = (g * 16) // _DIM
                    c = (g * 16) % _DIM
                    blk_v[d, b, l, pl.ds(c, 16)] = vals[i]
            return carry2

        lax.fori_loop(0, _K2_BB, bbody, 0)

    in_copy(0, 0).start()

    def body(blk, carry):
        for d in range(2):
            @pl.when(lax.rem(blk, 2) == d)
            def _():
                nd = 1 - d
                @pl.when(blk + 1 < _K2_NBLK)
                def _():
                    in_copy(blk + 1, nd).start()
                in_copy(blk, d).wait()
                @pl.when(blk >= 2)
                def _():
                    drain(blk - 2, d).wait()
                regcopy(d)
                drain(blk, d).start()
        return carry

    lax.fori_loop(0, _K2_NBLK, body, 0)

    drain(_K2_NBLK - 2, _K2_NBLK % 2).wait()
    drain(_K2_NBLK - 1, 1 - _K2_NBLK % 2).wait()


def kernel(x, table):
    flat = x.reshape(_B)
    tlin = _flatten_table(table)
    rows = _gather_rows(flat, tlin.reshape(_VOCAB, _DIM))
    out = _format_out(rows.reshape(_B * _DIM))
    return out[:, None, :, :]
